# baseline (speedup 1.0000x reference)
import functools

import jax
import jax.numpy as jnp
from jax import lax
from jax.experimental import pallas as pl
from jax.experimental.pallas import tpu as pltpu
from jax.experimental.pallas import tpu_sc as plsc

N = 10000
H = 128
HH = H // 2
L = 4
E = 320000

NC = 2
NS = 16
CHUNK = 128
EPT = 20480
EPAD = EPT * NS
CPT = EPT // CHUNK
NPAD = 10112
STRIPE = NPAD // NS
HB = CPT // 4


def _seg_sum_build():
  mesh = plsc.VectorSubcoreMesh(core_axis_name="c", subcore_axis_name="s")

  @functools.partial(
      pl.kernel,
      out_type=(jax.ShapeDtypeStruct((NPAD, HH), jnp.float32),
                jax.ShapeDtypeStruct((NPAD, HH), jnp.float32)),
      mesh=mesh,
      compiler_params=pltpu.CompilerParams(use_tc_tiling_on_sc=False),
      scratch_types=[
          pltpu.VMEM_SHARED((NPAD, HH), jnp.float32),
          pltpu.VMEM_SHARED((NPAD, HH), jnp.float32),
          pltpu.VMEM((HB, CHUNK), jnp.int32),
          pltpu.VMEM((HB, CHUNK), jnp.int32),
          pltpu.VMEM((2, CHUNK, HH), jnp.float32),
          pltpu.SemaphoreType.DMA,
          pltpu.SemaphoreType.DMA,
          pltpu.SemaphoreType.DMA,
          pltpu.SemaphoreType.DMA,
      ],
  )
  def seg_sum(h0_hbm, h1_hbm, src_hbm, dst_hbm, z_hbm, out0_hbm, out1_hbm,
              h_sp, acc, src_v, dst_v, rows_v, sem0, sem1, sem2, sem3):
    c = lax.axis_index("c")
    s = lax.axis_index("s")

    @pl.when(c == 0)
    def _():
      pltpu.sync_copy(h0_hbm.at[pl.ds(s * STRIPE, STRIPE)],
                      h_sp.at[pl.ds(s * STRIPE, STRIPE)])

    @pl.when(c == 1)
    def _():
      pltpu.sync_copy(h1_hbm.at[pl.ds(s * STRIPE, STRIPE)],
                      h_sp.at[pl.ds(s * STRIPE, STRIPE)])

    pltpu.sync_copy(z_hbm.at[pl.ds(s * STRIPE, STRIPE)],
                    acc.at[pl.ds(s * STRIPE, STRIPE)])
    plsc.subcore_barrier()

    sems = (sem0, sem1)
    ssems = (sem2, sem3)
    for batch in range(CPT // HB):
      pltpu.sync_copy(src_hbm.at[pl.ds(s * CPT + batch * HB, HB)], src_v)
      pltpu.sync_copy(dst_hbm.at[pl.ds(s * CPT + batch * HB, HB)], dst_v)
      for b in range(2):
        pltpu.async_copy(h_sp.at[src_v.at[b]], rows_v.at[b], sems[b])

      def body(i, carry):
        for b in range(2):
          k = 2 * i + b
          pltpu.make_async_copy(h_sp.at[src_v.at[b]], rows_v.at[b],
                                sems[b]).wait()
          pltpu.async_copy(rows_v.at[b], acc.at[dst_v.at[k]], ssems[b],
                           add=True)
        for b in range(2):
          k = 2 * i + b
          pltpu.make_async_copy(rows_v.at[b], acc.at[dst_v.at[k]],
                                ssems[b]).wait()
          pltpu.async_copy(h_sp.at[src_v.at[k + 2]], rows_v.at[b], sems[b])
        return carry

      lax.fori_loop(0, (HB - 2) // 2, body, 0)
      for b in range(2):
        k = HB - 2 + b
        pltpu.make_async_copy(h_sp.at[src_v.at[b]], rows_v.at[b],
                              sems[b]).wait()
        pltpu.async_copy(rows_v.at[b], acc.at[dst_v.at[k]], ssems[b],
                         add=True)
      for b in range(2):
        k = HB - 2 + b
        pltpu.make_async_copy(rows_v.at[b], acc.at[dst_v.at[k]],
                              ssems[b]).wait()

    plsc.subcore_barrier()

    @pl.when(c == 0)
    def _():
      pltpu.sync_copy(acc.at[pl.ds(s * STRIPE, STRIPE)],
                      out0_hbm.at[pl.ds(s * STRIPE, STRIPE)])

    @pl.when(c == 1)
    def _():
      pltpu.sync_copy(acc.at[pl.ds(s * STRIPE, STRIPE)],
                      out1_hbm.at[pl.ds(s * STRIPE, STRIPE)])

  return seg_sum


_seg_sum_cache = []


def _seg_sum(*args):
  if not _seg_sum_cache:
    _seg_sum_cache.append(_seg_sum_build())
  return _seg_sum_cache[0](*args)


def _bn(x, g, b):
  m = jnp.mean(x, axis=0, keepdims=True)
  v = jnp.mean((x - m) ** 2, axis=0, keepdims=True)
  return (x - m) * lax.rsqrt(v + 1e-5) * g + b


def _split_pad(x):
  z = jnp.zeros((NPAD - N, HH), x.dtype)
  return (jnp.concatenate([x[:, :HH], z], axis=0),
          jnp.concatenate([x[:, HH:], z], axis=0))


def _embed_body(h_ref, w_ref, b_ref, o0_ref, o1_ref):
  o = lax.dot_general(h_ref[...], w_ref[...], (((1,), (1,)), ((), ())),
                      preferred_element_type=jnp.float32) + b_ref[...]
  o0, o1 = _split_pad(o)
  o0_ref[...] = o0
  o1_ref[...] = o1


def _layer_math(h0_ref, h1_ref, q0_ref, q1_ref, w1_ref, b1_ref, g1_ref,
                bb1_ref, w2_ref, b2_ref, g2_ref, bb2_ref, g3_ref, bb3_ref):
  hv = jnp.concatenate([h0_ref[0:N, :], h1_ref[0:N, :]], axis=1)
  t = hv + jnp.concatenate([q0_ref[0:N, :], q1_ref[0:N, :]], axis=1)
  u = lax.dot_general(t, w1_ref[...], (((1,), (1,)), ((), ())),
                      preferred_element_type=jnp.float32) + b1_ref[...]
  u = jnp.maximum(_bn(u, g1_ref[...], bb1_ref[...]), 0.0)
  v = lax.dot_general(u, w2_ref[...], (((1,), (1,)), ((), ())),
                      preferred_element_type=jnp.float32) + b2_ref[...]
  v = jnp.maximum(_bn(v, g2_ref[...], bb2_ref[...]), 0.0)
  v = jnp.maximum(_bn(v, g3_ref[...], bb3_ref[...]), 0.0)
  return hv + v


def _dense_mid_body(h0_ref, h1_ref, q0_ref, q1_ref, w1_ref, b1_ref, g1_ref,
                    bb1_ref, w2_ref, b2_ref, g2_ref, bb2_ref, g3_ref, bb3_ref,
                    o0_ref, o1_ref):
  o = _layer_math(h0_ref, h1_ref, q0_ref, q1_ref, w1_ref, b1_ref, g1_ref,
                  bb1_ref, w2_ref, b2_ref, g2_ref, bb2_ref, g3_ref, bb3_ref)
  o0, o1 = _split_pad(o)
  o0_ref[...] = o0
  o1_ref[...] = o1


def _dense_last_body(h0_ref, h1_ref, q0_ref, q1_ref, w1_ref, b1_ref, g1_ref,
                     bb1_ref, w2_ref, b2_ref, g2_ref, bb2_ref, g3_ref,
                     bb3_ref, o_ref):
  o_ref[...] = _layer_math(h0_ref, h1_ref, q0_ref, q1_ref, w1_ref, b1_ref,
                           g1_ref, bb1_ref, w2_ref, b2_ref, g2_ref, bb2_ref,
                           g3_ref, bb3_ref)


_embed = pl.pallas_call(
    _embed_body,
    out_shape=(jax.ShapeDtypeStruct((NPAD, HH), jnp.float32),
               jax.ShapeDtypeStruct((NPAD, HH), jnp.float32)))

_dense_mid = pl.pallas_call(
    _dense_mid_body,
    out_shape=(jax.ShapeDtypeStruct((NPAD, HH), jnp.float32),
               jax.ShapeDtypeStruct((NPAD, HH), jnp.float32)))

_dense_last = pl.pallas_call(
    _dense_last_body,
    out_shape=jax.ShapeDtypeStruct((N, H), jnp.float32))


def kernel(h, edge_index, e, emb_W, emb_b, W1, b1, bn1_g, bn1_b,
           W2, b2, anf_g, anf_b, gin_g, gin_b):
  del e
  src = jnp.pad(edge_index[0], (0, EPAD - E))
  dst = jnp.pad(edge_index[1], (0, EPAD - E), constant_values=NPAD - 1)
  src2d = src.reshape(EPAD // CHUNK, CHUNK)
  dst2d = dst.reshape(EPAD // CHUNK, CHUNK)
  zeros = jnp.zeros((NPAD, HH), jnp.float32)

  h0, h1 = _embed(h, emb_W, emb_b.reshape(1, H))
  for l in range(L):
    q0, q1 = _seg_sum(h0, h1, src2d, dst2d, zeros)
    args = (W1[l], b1[l].reshape(1, H), bn1_g[l].reshape(1, H),
            bn1_b[l].reshape(1, H),
            W2[l], b2[l].reshape(1, H), anf_g[l].reshape(1, H),
            anf_b[l].reshape(1, H),
            gin_g[l].reshape(1, H), gin_b[l].reshape(1, H))
    if l < L - 1:
      h0, h1 = _dense_mid(h0, h1, q0, q1, *args)
    else:
      out = _dense_last(h0, h1, q0, q1, *args)
  return out


# speedup vs baseline: 1.0735x; 1.0735x over previous
import functools

import jax
import jax.numpy as jnp
from jax import lax
from jax.experimental import pallas as pl
from jax.experimental.pallas import tpu as pltpu
from jax.experimental.pallas import tpu_sc as plsc

N = 10000
H = 128
HH = H // 2
L = 4
E = 320000

NC = 2
NS = 16
CHUNK = 128
EPT = 20480
EPAD = EPT * NS
CPT = EPT // CHUNK
NPAD = 10112
STRIPE = NPAD // NS
HB = CPT // 4


def _seg_sum_build():
  mesh = plsc.VectorSubcoreMesh(core_axis_name="c", subcore_axis_name="s")

  @functools.partial(
      pl.kernel,
      out_type=(jax.ShapeDtypeStruct((NPAD, HH), jnp.float32),
                jax.ShapeDtypeStruct((NPAD, HH), jnp.float32)),
      mesh=mesh,
      compiler_params=pltpu.CompilerParams(use_tc_tiling_on_sc=False),
      scratch_types=[
          pltpu.VMEM_SHARED((NPAD, HH), jnp.float32),
          pltpu.VMEM_SHARED((NPAD, HH), jnp.float32),
          pltpu.VMEM((HB, CHUNK), jnp.int32),
          pltpu.VMEM((HB, CHUNK), jnp.int32),
          pltpu.VMEM((2, CHUNK, HH), jnp.float32),
          pltpu.SemaphoreType.DMA,
          pltpu.SemaphoreType.DMA,
      ],
  )
  def seg_sum(h0_hbm, h1_hbm, src_hbm, dst_hbm, z_hbm, out0_hbm, out1_hbm,
              h_sp, acc, src_v, dst_v, rows_v, sem0, sem1):
    c = lax.axis_index("c")
    s = lax.axis_index("s")

    @pl.when(c == 0)
    def _():
      pltpu.sync_copy(h0_hbm.at[pl.ds(s * STRIPE, STRIPE)],
                      h_sp.at[pl.ds(s * STRIPE, STRIPE)])

    @pl.when(c == 1)
    def _():
      pltpu.sync_copy(h1_hbm.at[pl.ds(s * STRIPE, STRIPE)],
                      h_sp.at[pl.ds(s * STRIPE, STRIPE)])

    pltpu.sync_copy(z_hbm.at[pl.ds(s * STRIPE, STRIPE)],
                    acc.at[pl.ds(s * STRIPE, STRIPE)])
    plsc.subcore_barrier()

    sems = (sem0, sem1)
    for batch in range(CPT // HB):
      pltpu.sync_copy(src_hbm.at[pl.ds(s * CPT + batch * HB, HB)], src_v)
      pltpu.sync_copy(dst_hbm.at[pl.ds(s * CPT + batch * HB, HB)], dst_v)
      for b in range(2):
        pltpu.async_copy(h_sp.at[src_v.at[b]], rows_v.at[b], sems[b])

      def body(i, carry):
        for b in range(2):
          k = 2 * i + b
          pltpu.make_async_copy(h_sp.at[src_v.at[b]], rows_v.at[b],
                                sems[b]).wait()
          pltpu.sync_copy(rows_v.at[b], acc.at[dst_v.at[k]], add=True)
          pltpu.async_copy(h_sp.at[src_v.at[k + 2]], rows_v.at[b], sems[b])
        return carry

      lax.fori_loop(0, (HB - 2) // 2, body, 0)
      for b in range(2):
        k = HB - 2 + b
        pltpu.make_async_copy(h_sp.at[src_v.at[b]], rows_v.at[b],
                              sems[b]).wait()
        pltpu.sync_copy(rows_v.at[b], acc.at[dst_v.at[k]], add=True)

    plsc.subcore_barrier()

    @pl.when(c == 0)
    def _():
      pltpu.sync_copy(acc.at[pl.ds(s * STRIPE, STRIPE)],
                      out0_hbm.at[pl.ds(s * STRIPE, STRIPE)])

    @pl.when(c == 1)
    def _():
      pltpu.sync_copy(acc.at[pl.ds(s * STRIPE, STRIPE)],
                      out1_hbm.at[pl.ds(s * STRIPE, STRIPE)])

  return seg_sum


_seg_sum_cache = []


def _seg_sum(*args):
  if not _seg_sum_cache:
    _seg_sum_cache.append(_seg_sum_build())
  return _seg_sum_cache[0](*args)


def _bn(x, g, b):
  m = jnp.mean(x, axis=0, keepdims=True)
  v = jnp.mean((x - m) ** 2, axis=0, keepdims=True)
  return (x - m) * lax.rsqrt(v + 1e-5) * g + b


def _split_pad(x):
  z = jnp.zeros((NPAD - N, HH), x.dtype)
  return (jnp.concatenate([x[:, :HH], z], axis=0),
          jnp.concatenate([x[:, HH:], z], axis=0))


def _embed_body(h_ref, w_ref, b_ref, o0_ref, o1_ref):
  o = lax.dot_general(h_ref[...], w_ref[...], (((1,), (1,)), ((), ())),
                      preferred_element_type=jnp.float32) + b_ref[...]
  o0, o1 = _split_pad(o)
  o0_ref[...] = o0
  o1_ref[...] = o1


def _layer_math(h0_ref, h1_ref, q0_ref, q1_ref, w1_ref, b1_ref, g1_ref,
                bb1_ref, w2_ref, b2_ref, g2_ref, bb2_ref, g3_ref, bb3_ref):
  hv = jnp.concatenate([h0_ref[0:N, :], h1_ref[0:N, :]], axis=1)
  t = hv + jnp.concatenate([q0_ref[0:N, :], q1_ref[0:N, :]], axis=1)
  u = lax.dot_general(t, w1_ref[...], (((1,), (1,)), ((), ())),
                      preferred_element_type=jnp.float32) + b1_ref[...]
  u = jnp.maximum(_bn(u, g1_ref[...], bb1_ref[...]), 0.0)
  v = lax.dot_general(u, w2_ref[...], (((1,), (1,)), ((), ())),
                      preferred_element_type=jnp.float32) + b2_ref[...]
  v = jnp.maximum(_bn(v, g2_ref[...], bb2_ref[...]), 0.0)
  v = jnp.maximum(_bn(v, g3_ref[...], bb3_ref[...]), 0.0)
  return hv + v


def _dense_mid_body(h0_ref, h1_ref, q0_ref, q1_ref, w1_ref, b1_ref, g1_ref,
                    bb1_ref, w2_ref, b2_ref, g2_ref, bb2_ref, g3_ref, bb3_ref,
                    o0_ref, o1_ref):
  o = _layer_math(h0_ref, h1_ref, q0_ref, q1_ref, w1_ref, b1_ref, g1_ref,
                  bb1_ref, w2_ref, b2_ref, g2_ref, bb2_ref, g3_ref, bb3_ref)
  o0, o1 = _split_pad(o)
  o0_ref[...] = o0
  o1_ref[...] = o1


def _dense_last_body(h0_ref, h1_ref, q0_ref, q1_ref, w1_ref, b1_ref, g1_ref,
                     bb1_ref, w2_ref, b2_ref, g2_ref, bb2_ref, g3_ref,
                     bb3_ref, o_ref):
  o_ref[...] = _layer_math(h0_ref, h1_ref, q0_ref, q1_ref, w1_ref, b1_ref,
                           g1_ref, bb1_ref, w2_ref, b2_ref, g2_ref, bb2_ref,
                           g3_ref, bb3_ref)


_embed = pl.pallas_call(
    _embed_body,
    out_shape=(jax.ShapeDtypeStruct((NPAD, HH), jnp.float32),
               jax.ShapeDtypeStruct((NPAD, HH), jnp.float32)))

_dense_mid = pl.pallas_call(
    _dense_mid_body,
    out_shape=(jax.ShapeDtypeStruct((NPAD, HH), jnp.float32),
               jax.ShapeDtypeStruct((NPAD, HH), jnp.float32)))

_dense_last = pl.pallas_call(
    _dense_last_body,
    out_shape=jax.ShapeDtypeStruct((N, H), jnp.float32))


def kernel(h, edge_index, e, emb_W, emb_b, W1, b1, bn1_g, bn1_b,
           W2, b2, anf_g, anf_b, gin_g, gin_b):
  del e
  src = jnp.pad(edge_index[0], (0, EPAD - E))
  dst = jnp.pad(edge_index[1], (0, EPAD - E), constant_values=NPAD - 1)
  src2d = src.reshape(EPAD // CHUNK, CHUNK)
  dst2d = dst.reshape(EPAD // CHUNK, CHUNK)
  zeros = jnp.zeros((NPAD, HH), jnp.float32)

  h0, h1 = _embed(h, emb_W, emb_b.reshape(1, H))
  for l in range(L):
    q0, q1 = _seg_sum(h0, h1, src2d, dst2d, zeros)
    args = (W1[l], b1[l].reshape(1, H), bn1_g[l].reshape(1, H),
            bn1_b[l].reshape(1, H),
            W2[l], b2[l].reshape(1, H), anf_g[l].reshape(1, H),
            anf_b[l].reshape(1, H),
            gin_g[l].reshape(1, H), gin_b[l].reshape(1, H))
    if l < L - 1:
      h0, h1 = _dense_mid(h0, h1, q0, q1, *args)
    else:
      out = _dense_last(h0, h1, q0, q1, *args)
  return out


# speedup vs baseline: 1.0913x; 1.0166x over previous
import functools

import jax
import jax.numpy as jnp
from jax import lax
from jax.experimental import pallas as pl
from jax.experimental.pallas import tpu as pltpu
from jax.experimental.pallas import tpu_sc as plsc

N = 10000
H = 128
HH = H // 2
L = 4
E = 320000

NC = 2
NS = 16
CHUNK = 128
EPT = 20480
EPAD = EPT * NS
CPT = EPT // CHUNK
NPAD = 10112
STRIPE = NPAD // NS
HB = CPT // 4


def _seg_sum_build():
  mesh = plsc.VectorSubcoreMesh(core_axis_name="c", subcore_axis_name="s")

  @functools.partial(
      pl.kernel,
      out_type=(jax.ShapeDtypeStruct((NPAD, HH), jnp.float32),
                jax.ShapeDtypeStruct((NPAD, HH), jnp.float32)),
      mesh=mesh,
      compiler_params=pltpu.CompilerParams(use_tc_tiling_on_sc=False),
      scratch_types=[
          pltpu.VMEM_SHARED((NPAD, HH), jnp.float32),
          pltpu.VMEM_SHARED((NPAD, HH), jnp.float32),
          pltpu.VMEM((HB, CHUNK), jnp.int32),
          pltpu.VMEM((HB, CHUNK), jnp.int32),
          pltpu.VMEM((2, CHUNK, HH), jnp.float32),
          pltpu.SemaphoreType.DMA,
          pltpu.SemaphoreType.DMA,
      ],
  )
  def seg_sum(h0_hbm, h1_hbm, src_hbm, dst_hbm, z_hbm, out0_hbm, out1_hbm,
              h_sp, acc, src_v, dst_v, rows_v, sem0, sem1):
    c = lax.axis_index("c")
    s = lax.axis_index("s")

    @pl.when(c == 0)
    def _():
      pltpu.sync_copy(h0_hbm.at[pl.ds(s * STRIPE, STRIPE)],
                      h_sp.at[pl.ds(s * STRIPE, STRIPE)])

    @pl.when(c == 1)
    def _():
      pltpu.sync_copy(h1_hbm.at[pl.ds(s * STRIPE, STRIPE)],
                      h_sp.at[pl.ds(s * STRIPE, STRIPE)])

    pltpu.sync_copy(z_hbm.at[pl.ds(s * STRIPE, STRIPE)],
                    acc.at[pl.ds(s * STRIPE, STRIPE)])
    plsc.subcore_barrier()

    sems = (sem0, sem1)
    for batch in range(CPT // HB):
      pltpu.sync_copy(src_hbm.at[pl.ds(s * CPT + batch * HB, HB)], src_v)
      pltpu.sync_copy(dst_hbm.at[pl.ds(s * CPT + batch * HB, HB)], dst_v)
      for b in range(2):
        pltpu.async_copy(h_sp.at[src_v.at[b]], rows_v.at[b], sems[b])

      def body(i, carry):
        for b in range(2):
          k = 2 * i + b
          pltpu.make_async_copy(h_sp.at[src_v.at[b]], rows_v.at[b],
                                sems[b]).wait()
          pltpu.sync_copy(rows_v.at[b], acc.at[dst_v.at[k]], add=True)
          pltpu.async_copy(h_sp.at[src_v.at[k + 2]], rows_v.at[b], sems[b])
        return carry

      lax.fori_loop(0, (HB - 2) // 2, body, 0)
      for b in range(2):
        k = HB - 2 + b
        pltpu.make_async_copy(h_sp.at[src_v.at[b]], rows_v.at[b],
                              sems[b]).wait()
        pltpu.sync_copy(rows_v.at[b], acc.at[dst_v.at[k]], add=True)

    plsc.subcore_barrier()

    @pl.when(c == 0)
    def _():
      pltpu.sync_copy(acc.at[pl.ds(s * STRIPE, STRIPE)],
                      out0_hbm.at[pl.ds(s * STRIPE, STRIPE)])

    @pl.when(c == 1)
    def _():
      pltpu.sync_copy(acc.at[pl.ds(s * STRIPE, STRIPE)],
                      out1_hbm.at[pl.ds(s * STRIPE, STRIPE)])

  return seg_sum


_seg_sum_cache = []


def _seg_sum(*args):
  if not _seg_sum_cache:
    _seg_sum_cache.append(_seg_sum_build())
  return _seg_sum_cache[0](*args)


def _bn(x, g, b):
  m = jnp.mean(x, axis=0, keepdims=True)
  m2 = jnp.mean(x * x, axis=0, keepdims=True)
  v = m2 - m * m
  return (x - m) * lax.rsqrt(v + 1e-5) * g + b


def _split_pad(x):
  z = jnp.zeros((NPAD - N, HH), x.dtype)
  return (jnp.concatenate([x[:, :HH], z], axis=0),
          jnp.concatenate([x[:, HH:], z], axis=0))


def _embed_body(h_ref, w_ref, b_ref, o0_ref, o1_ref):
  o = lax.dot_general(h_ref[...], w_ref[...], (((1,), (1,)), ((), ())),
                      preferred_element_type=jnp.float32) + b_ref[...]
  o0, o1 = _split_pad(o)
  o0_ref[...] = o0
  o1_ref[...] = o1


def _layer_math(h0_ref, h1_ref, q0_ref, q1_ref, w1_ref, b1_ref, g1_ref,
                bb1_ref, w2_ref, b2_ref, g2_ref, bb2_ref, g3_ref, bb3_ref):
  hv = jnp.concatenate([h0_ref[0:N, :], h1_ref[0:N, :]], axis=1)
  t = hv + jnp.concatenate([q0_ref[0:N, :], q1_ref[0:N, :]], axis=1)
  u = lax.dot_general(t, w1_ref[...], (((1,), (1,)), ((), ())),
                      preferred_element_type=jnp.float32) + b1_ref[...]
  u = jnp.maximum(_bn(u, g1_ref[...], bb1_ref[...]), 0.0)
  v = lax.dot_general(u, w2_ref[...], (((1,), (1,)), ((), ())),
                      preferred_element_type=jnp.float32) + b2_ref[...]
  v = jnp.maximum(_bn(v, g2_ref[...], bb2_ref[...]), 0.0)
  v = jnp.maximum(_bn(v, g3_ref[...], bb3_ref[...]), 0.0)
  return hv + v


def _dense_mid_body(h0_ref, h1_ref, q0_ref, q1_ref, w1_ref, b1_ref, g1_ref,
                    bb1_ref, w2_ref, b2_ref, g2_ref, bb2_ref, g3_ref, bb3_ref,
                    o0_ref, o1_ref):
  o = _layer_math(h0_ref, h1_ref, q0_ref, q1_ref, w1_ref, b1_ref, g1_ref,
                  bb1_ref, w2_ref, b2_ref, g2_ref, bb2_ref, g3_ref, bb3_ref)
  o0, o1 = _split_pad(o)
  o0_ref[...] = o0
  o1_ref[...] = o1


def _dense_last_body(h0_ref, h1_ref, q0_ref, q1_ref, w1_ref, b1_ref, g1_ref,
                     bb1_ref, w2_ref, b2_ref, g2_ref, bb2_ref, g3_ref,
                     bb3_ref, o_ref):
  o_ref[...] = _layer_math(h0_ref, h1_ref, q0_ref, q1_ref, w1_ref, b1_ref,
                           g1_ref, bb1_ref, w2_ref, b2_ref, g2_ref, bb2_ref,
                           g3_ref, bb3_ref)


_embed = pl.pallas_call(
    _embed_body,
    out_shape=(jax.ShapeDtypeStruct((NPAD, HH), jnp.float32),
               jax.ShapeDtypeStruct((NPAD, HH), jnp.float32)))

_dense_mid = pl.pallas_call(
    _dense_mid_body,
    out_shape=(jax.ShapeDtypeStruct((NPAD, HH), jnp.float32),
               jax.ShapeDtypeStruct((NPAD, HH), jnp.float32)))

_dense_last = pl.pallas_call(
    _dense_last_body,
    out_shape=jax.ShapeDtypeStruct((N, H), jnp.float32))


def kernel(h, edge_index, e, emb_W, emb_b, W1, b1, bn1_g, bn1_b,
           W2, b2, anf_g, anf_b, gin_g, gin_b):
  del e
  src = jnp.pad(edge_index[0], (0, EPAD - E))
  dst = jnp.pad(edge_index[1], (0, EPAD - E), constant_values=NPAD - 1)
  src2d = src.reshape(EPAD // CHUNK, CHUNK)
  dst2d = dst.reshape(EPAD // CHUNK, CHUNK)
  zeros = jnp.zeros((NPAD, HH), jnp.float32)

  h0, h1 = _embed(h, emb_W, emb_b.reshape(1, H))
  for l in range(L):
    q0, q1 = _seg_sum(h0, h1, src2d, dst2d, zeros)
    args = (W1[l], b1[l].reshape(1, H), bn1_g[l].reshape(1, H),
            bn1_b[l].reshape(1, H),
            W2[l], b2[l].reshape(1, H), anf_g[l].reshape(1, H),
            anf_b[l].reshape(1, H),
            gin_g[l].reshape(1, H), gin_b[l].reshape(1, H))
    if l < L - 1:
      h0, h1 = _dense_mid(h0, h1, q0, q1, *args)
    else:
      out = _dense_last(h0, h1, q0, q1, *args)
  return out
